# baseline (speedup 1.0000x reference)
import jax
import jax.numpy as jnp
import numpy as np
from jax import lax
from jax.experimental import pallas as pl
from jax.experimental.pallas import tpu as pltpu, tpu_sc as plsc

import functools


_LANES = 16


def _make_sc_gather(n_layers, B, S, D):
    n_rows = 3 * B
    assert n_rows <= _LANES and D % 128 == 0
    base = (n_layers - 1) * B * S

    lanes = np.arange(_LANES)
    bat = np.minimum(lanes // 3, B - 1)
    base_np = (base + bat * S).astype(np.int32)

    sl = D // 128
    mesh = plsc.VectorSubcoreMesh(core_axis_name="c", subcore_axis_name="s")

    @functools.partial(
        pl.kernel,
        mesh=mesh,
        out_type=jax.ShapeDtypeStruct((n_rows, sl, 128), jnp.float32),
        scratch_types=[
            pltpu.VMEM((2 * _LANES,), jnp.int32),
            pltpu.VMEM((_LANES,), jnp.int32),
            pltpu.VMEM((n_rows, sl, 128), jnp.float32),
            pltpu.SemaphoreType.DMA,
        ],
    )
    def sc_gather(table_hbm, posbase_hbm, out_hbm, posbase_v, row_idx,
                  rows, sem):
        wid = lax.axis_index("s") * 2 + lax.axis_index("c")

        @pl.when(wid == 0)
        def _():
            pltpu.sync_copy(posbase_hbm, posbase_v)
            pos = posbase_v[pl.ds(0, _LANES)]
            basev = posbase_v[pl.ds(_LANES, _LANES)]
            row_idx[...] = basev + pos
            pltpu.async_copy(
                table_hbm.at[row_idx.at[pl.ds(0, n_rows)]], rows, sem
            ).wait()
            pltpu.sync_copy(rows, out_hbm)

    return sc_gather, base_np


def kernel(feature, idx1, idx2):
    n_layers, B, S, D = feature.shape
    table = feature.reshape(n_layers * B * S, D // 128, 128)
    zero = jnp.zeros_like(idx1, dtype=jnp.int32)
    pos = jnp.stack([zero, idx1.astype(jnp.int32), idx2.astype(jnp.int32)],
                    axis=1).reshape(-1)
    pos = jnp.concatenate([pos, jnp.zeros((_LANES - 3 * B,), jnp.int32)])
    sc_gather, base_np = _make_sc_gather(n_layers, B, S, D)
    posbase = jnp.concatenate([pos, jnp.asarray(base_np)])
    out = sc_gather(table, posbase)
    return out.reshape(B, 3 * D)


# speedup vs baseline: 3.0422x; 3.0422x over previous
import jax
import jax.numpy as jnp
import numpy as np
from jax import lax
from jax.experimental import pallas as pl
from jax.experimental.pallas import tpu as pltpu, tpu_sc as plsc

import functools


_LANES = 16


def _make_sc_gather(n_layers, B, S, D):
    assert B == 4
    base = (n_layers - 1) * B * S
    lanes = np.arange(_LANES)
    bat = np.where(lanes < 3 * B, lanes % B, 0)
    base_np = (base + bat * S).astype(np.int32)

    mesh = plsc.VectorSubcoreMesh(core_axis_name="c", subcore_axis_name="s")

    @functools.partial(
        pl.kernel,
        mesh=mesh,
        out_type=jax.ShapeDtypeStruct((_LANES, D), jnp.float32),
        scratch_types=[
            pltpu.VMEM((_LANES,), jnp.int32),
            pltpu.VMEM((_LANES,), jnp.int32),
            pltpu.VMEM((_LANES,), jnp.int32),
            pltpu.VMEM((_LANES, D), jnp.float32),
            pltpu.SemaphoreType.DMA,
        ],
    )
    def sc_gather(table_hbm, idx1_hbm, idx2_hbm, base_hbm, out_hbm,
                  pos_v, base_v, row_idx, rows, sem):
        wid = lax.axis_index("s") * 2 + lax.axis_index("c")

        @pl.when(wid == 0)
        def _():
            pos_v[...] = jnp.zeros((_LANES,), jnp.int32)
            pltpu.sync_copy(idx1_hbm, pos_v.at[pl.ds(0, B)])
            pltpu.sync_copy(idx2_hbm, pos_v.at[pl.ds(2 * B, B)])
            pltpu.sync_copy(base_hbm, base_v)
            row_idx[...] = base_v[...] + pos_v[...]
            pltpu.async_copy(table_hbm.at[row_idx], rows, sem).wait()
            pltpu.sync_copy(rows, out_hbm)

    return sc_gather, base_np


def kernel(feature, idx1, idx2):
    n_layers, B, S, D = feature.shape
    table = feature.reshape(n_layers * B * S, D)
    sc_gather, base_np = _make_sc_gather(n_layers, B, S, D)
    rows = sc_gather(table, idx1.astype(jnp.int32), idx2.astype(jnp.int32),
                     jnp.asarray(base_np))
    return jnp.concatenate(
        [rows[B:2 * B], rows[:B], rows[2 * B:3 * B]], axis=1)


# speedup vs baseline: 3.3049x; 1.0864x over previous
import jax
import jax.numpy as jnp
import numpy as np
from jax import lax
from jax.experimental import pallas as pl
from jax.experimental.pallas import tpu as pltpu, tpu_sc as plsc

import functools


_LANES = 16


def _make_sc_gather(n_layers, B, S, D):
    assert 2 * B <= _LANES and D % 128 == 0
    base = (n_layers - 1) * B * S
    lanes = np.arange(_LANES)
    bat = lanes % B
    base_a = (base + bat * S).astype(np.int32)
    base_b = np.where((lanes >= 8) & (lanes < 8 + B),
                      base + bat * S, base).astype(np.int32)
    base_np = np.concatenate([base_a, base_b])

    mesh = plsc.VectorSubcoreMesh(core_axis_name="c", subcore_axis_name="s")

    @functools.partial(
        pl.kernel,
        mesh=mesh,
        out_type=jax.ShapeDtypeStruct((B, 3 * D), jnp.float32),
        scratch_types=[
            pltpu.VMEM((_LANES,), jnp.int32),
            pltpu.VMEM((_LANES,), jnp.int32),
            pltpu.VMEM((2 * _LANES,), jnp.int32),
            pltpu.VMEM((_LANES,), jnp.int32),
            pltpu.VMEM((_LANES,), jnp.int32),
            pltpu.VMEM((B, D), jnp.float32),
            pltpu.VMEM((B, D), jnp.float32),
            pltpu.VMEM((B, D), jnp.float32),
            pltpu.SemaphoreType.DMA,
        ],
    )
    def sc_gather(table_hbm, idx1_hbm, idx2_hbm, base_hbm, out_hbm,
                  pos_a, pos_b, base_v, ridx_a, ridx_b,
                  cls_v, w1_v, w2_v, sem):
        wid = lax.axis_index("s") * 2 + lax.axis_index("c")

        @pl.when(wid == 0)
        def _():
            pos_a[...] = jnp.zeros((_LANES,), jnp.int32)
            pos_b[...] = jnp.zeros((_LANES,), jnp.int32)
            pltpu.sync_copy(idx1_hbm, pos_a.at[pl.ds(8, B)])
            pltpu.sync_copy(idx2_hbm, pos_b.at[pl.ds(8, B)])
            pltpu.sync_copy(base_hbm, base_v)
            ridx_a[...] = base_v[pl.ds(0, _LANES)] + pos_a[...]
            ridx_b[...] = base_v[pl.ds(_LANES, _LANES)] + pos_b[...]

            g1 = pltpu.async_copy(
                table_hbm.at[ridx_a.at[pl.ds(0, B)]], cls_v, sem)
            g2 = pltpu.async_copy(
                table_hbm.at[ridx_a.at[pl.ds(8, B)]], w1_v, sem)
            g3 = pltpu.async_copy(
                table_hbm.at[ridx_b.at[pl.ds(8, B)]], w2_v, sem)
            g1.wait()
            g2.wait()
            g3.wait()

            o1 = pltpu.async_copy(cls_v, out_hbm.at[:, pl.ds(0, D)], sem)
            o2 = pltpu.async_copy(w1_v, out_hbm.at[:, pl.ds(D, D)], sem)
            o3 = pltpu.async_copy(w2_v, out_hbm.at[:, pl.ds(2 * D, D)], sem)
            o1.wait()
            o2.wait()
            o3.wait()

    return sc_gather, base_np


def kernel(feature, idx1, idx2):
    n_layers, B, S, D = feature.shape
    table = feature.reshape(n_layers * B * S, D)
    sc_gather, base_np = _make_sc_gather(n_layers, B, S, D)
    return sc_gather(table, idx1.astype(jnp.int32), idx2.astype(jnp.int32),
                     jnp.asarray(base_np))


# speedup vs baseline: 3.7364x; 1.1305x over previous
import jax
import jax.numpy as jnp
import numpy as np
from jax import lax
from jax.experimental import pallas as pl
from jax.experimental.pallas import tpu as pltpu, tpu_sc as plsc

import functools


_LANES = 16


def _make_sc_gather(n_layers, B, S, D):
    assert 8 + B <= _LANES and D % 128 == 0
    base = (n_layers - 1) * B * S
    lanes = np.arange(_LANES)
    bat = lanes % B
    col_base = np.where((lanes >= 8) & (lanes < 8 + B),
                        base + bat * S, base).astype(np.int32)
    base_np = np.tile(col_base, 3)

    mesh = plsc.VectorSubcoreMesh(core_axis_name="c", subcore_axis_name="s",
                                  num_cores=1)

    @functools.partial(
        pl.kernel,
        mesh=mesh,
        out_type=jax.ShapeDtypeStruct((B, 3 * D), jnp.float32),
        scratch_types=[
            pltpu.VMEM((_LANES,), jnp.int32),
            pltpu.VMEM((_LANES,), jnp.int32),
            pltpu.VMEM((_LANES,), jnp.int32),
            pltpu.VMEM((B, D), jnp.float32),
            pltpu.SemaphoreType.DMA,
        ],
    )
    def sc_gather(table_hbm, idx1_hbm, idx2_hbm, base_hbm, out_hbm,
                  pos_v, base_v, ridx, rows, sem):
        tid = lax.axis_index("s")

        def column(col, idx_hbm):
            pos_v[...] = jnp.zeros((_LANES,), jnp.int32)
            if idx_hbm is not None:
                pltpu.sync_copy(idx_hbm, pos_v.at[pl.ds(8, B)])
            pltpu.sync_copy(base_hbm.at[pl.ds(col * _LANES, _LANES)], base_v)
            ridx[...] = base_v[...] + pos_v[...]
            pltpu.async_copy(
                table_hbm.at[ridx.at[pl.ds(8, B)]], rows, sem).wait()
            pltpu.sync_copy(rows, out_hbm.at[:, pl.ds(col * D, D)])

        @pl.when(tid == 0)
        def _():
            column(0, None)

        @pl.when(tid == 1)
        def _():
            column(1, idx1_hbm)

        @pl.when(tid == 2)
        def _():
            column(2, idx2_hbm)

    return sc_gather, base_np


def kernel(feature, idx1, idx2):
    n_layers, B, S, D = feature.shape
    table = feature.reshape(n_layers * B * S, D)
    sc_gather, base_np = _make_sc_gather(n_layers, B, S, D)
    return sc_gather(table, idx1.astype(jnp.int32), idx2.astype(jnp.int32),
                     jnp.asarray(base_np))


# speedup vs baseline: 3.7956x; 1.0158x over previous
import jax
import jax.numpy as jnp
from jax import lax
from jax.experimental import pallas as pl
from jax.experimental.pallas import tpu as pltpu, tpu_sc as plsc

import functools


_LANES = 16


def _make_sc_gather(n_layers, B, S, D):
    assert 8 + B <= _LANES and D % 128 == 0
    base = (n_layers - 1) * B * S

    mesh = plsc.VectorSubcoreMesh(core_axis_name="c", subcore_axis_name="s",
                                  num_cores=1)

    @functools.partial(
        pl.kernel,
        mesh=mesh,
        out_type=jax.ShapeDtypeStruct((B, 3 * D), jnp.float32),
        scratch_types=[
            pltpu.VMEM((_LANES,), jnp.int32),
            pltpu.VMEM((_LANES,), jnp.int32),
            pltpu.VMEM((B, D), jnp.float32),
            pltpu.SemaphoreType.DMA,
        ],
    )
    def sc_gather(table_hbm, idx1_hbm, idx2_hbm, out_hbm,
                  pos_v, ridx, rows, sem):
        tid = lax.axis_index("s")
        ramp = base + (lax.iota(jnp.int32, _LANES) - 8) * S

        def column(col, idx_hbm):
            if idx_hbm is not None:
                pltpu.sync_copy(idx_hbm, pos_v.at[pl.ds(8, B)])
                ridx[...] = ramp + pos_v[...]
            else:
                ridx[...] = ramp
            pltpu.async_copy(
                table_hbm.at[ridx.at[pl.ds(8, B)]], rows, sem).wait()
            pltpu.sync_copy(rows, out_hbm.at[:, pl.ds(col * D, D)])

        @pl.when(tid == 0)
        def _():
            column(0, None)

        @pl.when(tid == 1)
        def _():
            column(1, idx1_hbm)

        @pl.when(tid == 2)
        def _():
            column(2, idx2_hbm)

    return sc_gather


def kernel(feature, idx1, idx2):
    n_layers, B, S, D = feature.shape
    table = feature.reshape(n_layers * B * S, D)
    sc_gather = _make_sc_gather(n_layers, B, S, D)
    return sc_gather(table, idx1.astype(jnp.int32), idx2.astype(jnp.int32))


# speedup vs baseline: 3.8088x; 1.0035x over previous
import jax
import jax.numpy as jnp
from jax import lax
from jax.experimental import pallas as pl
from jax.experimental.pallas import tpu as pltpu, tpu_sc as plsc

import functools


_LANES = 16


def _make_sc_gather(n_layers, B, S, D):
    assert 8 + B <= _LANES and D % 128 == 0
    base = (n_layers - 1) * B * S

    mesh = plsc.VectorSubcoreMesh(core_axis_name="c", subcore_axis_name="s",
                                  num_cores=1, num_subcores=3)

    @functools.partial(
        pl.kernel,
        mesh=mesh,
        out_type=jax.ShapeDtypeStruct((B, 3 * D), jnp.float32),
        scratch_types=[
            pltpu.VMEM((_LANES,), jnp.int32),
            pltpu.VMEM((_LANES,), jnp.int32),
            pltpu.VMEM((B, D), jnp.float32),
            pltpu.SemaphoreType.DMA,
        ],
    )
    def sc_gather(table_hbm, idx1_hbm, idx2_hbm, out_hbm,
                  pos_v, ridx, rows, sem):
        tid = lax.axis_index("s")
        ramp = base + (lax.iota(jnp.int32, _LANES) - 8) * S

        def column(col, idx_hbm):
            if idx_hbm is not None:
                pltpu.sync_copy(idx_hbm, pos_v.at[pl.ds(8, B)])
                ridx[...] = ramp + pos_v[...]
            else:
                ridx[...] = ramp
            pltpu.async_copy(
                table_hbm.at[ridx.at[pl.ds(8, B)]], rows, sem).wait()
            pltpu.sync_copy(rows, out_hbm.at[:, pl.ds(col * D, D)])

        @pl.when(tid == 0)
        def _():
            column(0, None)

        @pl.when(tid == 1)
        def _():
            column(1, idx1_hbm)

        @pl.when(tid == 2)
        def _():
            column(2, idx2_hbm)

    return sc_gather


def kernel(feature, idx1, idx2):
    n_layers, B, S, D = feature.shape
    table = feature.reshape(n_layers * B * S, D)
    sc_gather = _make_sc_gather(n_layers, B, S, D)
    return sc_gather(table, idx1.astype(jnp.int32), idx2.astype(jnp.int32))
